# baseline (speedup 1.0000x reference)
import functools

import jax
import jax.numpy as jnp
from jax import lax
from jax.experimental import pallas as pl
from jax.experimental.pallas import tpu as pltpu
from jax.experimental.pallas import tpu_sc as plsc

_ROWS = 8192
_COLS = 4
_WORDS = _ROWS * _COLS


@functools.partial(jax.jit, static_argnums=())
def _sc_permute_scatter(idx16, update_flat):
    info = plsc.get_sparse_core_info()
    nc, ns, lanes = info.num_cores, info.num_subcores, info.num_lanes
    nw = nc * ns
    wpw = _WORDS // nw
    chunks = wpw // lanes

    mesh = plsc.VectorSubcoreMesh(core_axis_name="c", subcore_axis_name="s")

    @functools.partial(
        pl.kernel,
        mesh=mesh,
        out_type=jax.ShapeDtypeStruct((_WORDS,), jnp.float32),
        scratch_types=[
            pltpu.VMEM((lanes,), jnp.int32),
            pltpu.VMEM((wpw,), jnp.float32),
            pltpu.VMEM((wpw,), jnp.float32),
        ],
        compiler_params=pltpu.CompilerParams(needs_layout_passes=False),
    )
    def k(idx_hbm, upd_hbm, out_hbm, idx_v, in_v, out_v):
        wid = lax.axis_index("s") * nc + lax.axis_index("c")
        base = wid * wpw
        pltpu.sync_copy(idx_hbm, idx_v)
        pltpu.sync_copy(upd_hbm.at[pl.ds(base, wpw)], in_v)
        lane = lax.iota(jnp.int32, lanes)
        tgt0 = (lane // _COLS) * _COLS + idx_v[...]
        for c in range(chunks):
            data = in_v[pl.ds(c * lanes, lanes)]
            plsc.store_scatter(out_v, [tgt0 + c * lanes], data)
        pltpu.sync_copy(out_v, out_hbm.at[pl.ds(base, wpw)])

    return k(idx16, update_flat)


def kernel(index, update, params):
    idx16 = jnp.tile(index.astype(jnp.int32), 16 // _COLS)
    out_flat = _sc_permute_scatter(idx16, update.reshape(-1))
    return out_flat.reshape(params.shape)


# speedup vs baseline: 1.0075x; 1.0075x over previous
import functools

import jax
import jax.numpy as jnp
from jax import lax
from jax.experimental import pallas as pl
from jax.experimental.pallas import tpu as pltpu
from jax.experimental.pallas import tpu_sc as plsc

_ROWS = 8192
_COLS = 4
_LANES = 16


def _sc_col_scatter(idx16, update):
    info = plsc.get_sparse_core_info()
    nc, ns = info.num_cores, info.num_subcores
    nw = nc * ns
    rpw = _ROWS // nw
    chunks = rpw * _COLS // _LANES

    mesh = plsc.VectorSubcoreMesh(core_axis_name="c", subcore_axis_name="s")

    @functools.partial(
        pl.kernel,
        mesh=mesh,
        out_type=jax.ShapeDtypeStruct((1, _ROWS, _COLS), jnp.float32),
        scratch_types=[
            pltpu.VMEM((_LANES,), jnp.int32),
            pltpu.VMEM((rpw, _COLS), jnp.float32),
            pltpu.VMEM((rpw, _COLS), jnp.float32),
        ],
        compiler_params=pltpu.CompilerParams(
            use_tc_tiling_on_sc=False, needs_layout_passes=False),
    )
    def k(idx_hbm, upd_hbm, out_hbm, idx_v, in_v, out_v):
        wid = lax.axis_index("s") * nc + lax.axis_index("c")
        rows = pl.ds(wid * rpw, rpw)
        pltpu.sync_copy(idx_hbm, idx_v)
        pltpu.sync_copy(upd_hbm.at[rows], in_v)
        lane = lax.iota(jnp.int32, _LANES)
        r0 = lane // _COLS
        src_c = lane % _COLS
        dst_c = idx_v[...]
        for c in range(chunks):
            r = r0 + c * (_LANES // _COLS)
            data = plsc.load_gather(in_v, [r, src_c])
            plsc.store_scatter(out_v, [r, dst_c], data)
        pltpu.sync_copy(out_v, out_hbm.at[0, rows])

    return k(idx16, update)


def kernel(index, update, params):
    del params
    idx16 = jnp.tile(index.astype(jnp.int32), _LANES // _COLS)
    return _sc_col_scatter(idx16, update)


# speedup vs baseline: 1.1660x; 1.1573x over previous
import functools

import jax
import jax.numpy as jnp
from jax import lax
from jax.experimental import pallas as pl
from jax.experimental.pallas import tpu as pltpu
from jax.experimental.pallas import tpu_sc as plsc

_ROWS = 8192
_COLS = 4
_LANES = 16


def _sc_col_scatter(index, update):
    info = plsc.get_sparse_core_info()
    nc, ns = info.num_cores, info.num_subcores
    nw = nc * ns
    rpw = _ROWS // nw
    chunks = rpw * _COLS // _LANES

    mesh = plsc.VectorSubcoreMesh(core_axis_name="c", subcore_axis_name="s")

    @functools.partial(
        pl.kernel,
        mesh=mesh,
        out_type=jax.ShapeDtypeStruct((1, _ROWS, _COLS), jnp.float32),
        scratch_types=[
            pltpu.VMEM((_COLS,), jnp.int32),
            pltpu.VMEM((rpw, _COLS), jnp.float32),
            pltpu.VMEM((rpw, _COLS), jnp.float32),
            pltpu.SemaphoreType.DMA,
        ],
        compiler_params=pltpu.CompilerParams(needs_layout_passes=False),
    )
    def k(idx_hbm, upd_hbm, out_hbm, idx_v, in_v, out_v, sem):
        wid = lax.axis_index("s") * nc + lax.axis_index("c")
        rows = pl.ds(wid * rpw, rpw)
        pltpu.async_copy(idx_hbm, idx_v, sem)
        pltpu.async_copy(upd_hbm.at[rows], in_v, sem)
        pltpu.make_async_copy(idx_hbm, idx_v, sem).wait()
        pltpu.make_async_copy(upd_hbm.at[rows], in_v, sem).wait()
        lane = lax.iota(jnp.int32, _LANES)
        r0 = lane // _COLS
        src_c = lane % _COLS
        dst_c = plsc.load_gather(idx_v, [src_c])
        for c in range(chunks):
            r = r0 + c * (_LANES // _COLS)
            data = plsc.load_gather(in_v, [r, src_c])
            plsc.store_scatter(out_v, [r, dst_c], data)
        pltpu.sync_copy(out_v, out_hbm.at[0, rows])

    return k(index, update)


def kernel(index, update, params):
    del params
    return _sc_col_scatter(index.astype(jnp.int32), update)


# speedup vs baseline: 1.1720x; 1.0051x over previous
import functools

import jax
import jax.numpy as jnp
from jax import lax
from jax.experimental import pallas as pl
from jax.experimental.pallas import tpu as pltpu
from jax.experimental.pallas import tpu_sc as plsc

_ROWS = 8192
_COLS = 4
_LANES = 16


def _sc_col_scatter(index, update):
    info = plsc.get_sparse_core_info()
    nc, ns = info.num_cores, info.num_subcores
    nw = nc * ns
    rpw = _ROWS // nw
    chunks = rpw * _COLS // _LANES

    mesh = plsc.VectorSubcoreMesh(core_axis_name="c", subcore_axis_name="s",
                                  num_cores=nc)

    @functools.partial(
        pl.kernel,
        mesh=mesh,
        out_type=jax.ShapeDtypeStruct((1, _ROWS, _COLS), jnp.float32),
        scratch_types=[
            pltpu.VMEM((_COLS,), jnp.int32),
            pltpu.VMEM((rpw, _COLS), jnp.float32),
            pltpu.VMEM((rpw, _COLS), jnp.float32),
            pltpu.SemaphoreType.DMA,
        ],
        compiler_params=pltpu.CompilerParams(needs_layout_passes=False),
    )
    def k(idx_hbm, upd_hbm, out_hbm, idx_v, in_v, out_v, sem):
        wid = lax.axis_index("s") * nc + lax.axis_index("c")
        rows = pl.ds(wid * rpw, rpw)
        pltpu.async_copy(idx_hbm, idx_v, sem)
        pltpu.async_copy(upd_hbm.at[rows], in_v, sem)
        pltpu.make_async_copy(idx_hbm, idx_v, sem).wait()
        pltpu.make_async_copy(upd_hbm.at[rows], in_v, sem).wait()
        lane = lax.iota(jnp.int32, _LANES)
        r0 = lane // _COLS
        src_c = lane % _COLS
        dst_c = plsc.load_gather(idx_v, [src_c])
        for c in range(chunks):
            r = r0 + c * (_LANES // _COLS)
            data = plsc.load_gather(in_v, [r, src_c])
            plsc.store_scatter(out_v, [r, dst_c], data)
        pltpu.sync_copy(out_v, out_hbm.at[0, rows])

    return k(index, update)


def kernel(index, update, params):
    del params
    return _sc_col_scatter(index.astype(jnp.int32), update)
